# baseline (speedup 1.0000x reference)
import functools

import jax
import jax.numpy as jnp
from jax import lax
from jax.experimental import pallas as pl
from jax.experimental.pallas import tpu as pltpu
from jax.experimental.pallas import tpu_sc as plsc


def _make_gather(V, D, B):
    info = plsc.get_sparse_core_info()
    NC, NS = info.num_cores, info.num_subcores
    NW = NC * NS
    assert B % (8 * NW) == 0
    b_per_w = B // NW
    mesh = plsc.VectorSubcoreMesh(core_axis_name="c", subcore_axis_name="s")

    @functools.partial(
        pl.kernel,
        mesh=mesh,
        out_type=jax.ShapeDtypeStruct((B, D), jnp.float32),
        scratch_types=[
            pltpu.VMEM((b_per_w,), jnp.int32),
            pltpu.VMEM((b_per_w, D), jnp.float32),
            pltpu.SemaphoreType.DMA,
        ],
        compiler_params=pltpu.CompilerParams(use_tc_tiling_on_sc=False),
    )
    def k(table_hbm, idx_hbm, out_hbm, idx_v, rows_v, sem):
        wid = lax.axis_index("s") * NC + lax.axis_index("c")
        base = wid * b_per_w
        pltpu.sync_copy(idx_hbm.at[pl.ds(base, b_per_w)], idx_v)
        pltpu.async_copy(table_hbm.at[idx_v], rows_v, sem).wait()
        pltpu.sync_copy(rows_v, out_hbm.at[pl.ds(base, b_per_w)])

    return k


def kernel(target, table):
    B = target.shape[0]
    V, D = table.shape
    k = _make_gather(V, D, B)
    return k(table, target.astype(jnp.int32))


# speedup vs baseline: 1.0239x; 1.0239x over previous
import functools

import jax
import jax.numpy as jnp
from jax import lax
from jax.experimental import pallas as pl
from jax.experimental.pallas import tpu as pltpu
from jax.experimental.pallas import tpu_sc as plsc


def _make_gather(V, D, B):
    info = plsc.get_sparse_core_info()
    NC, NS = info.num_cores, info.num_subcores
    NW = NC * NS
    assert B % (8 * NW) == 0
    b_per_w = B // NW
    K = 64
    mesh = plsc.VectorSubcoreMesh(core_axis_name="c", subcore_axis_name="s")

    @functools.partial(
        pl.kernel,
        mesh=mesh,
        out_type=jax.ShapeDtypeStruct((B, D), jnp.float32),
        scratch_types=[
            pltpu.VMEM((b_per_w,), jnp.int32),
            pltpu.SemaphoreType.DMA,
        ],
        compiler_params=pltpu.CompilerParams(needs_layout_passes=False),
    )
    def k(table_hbm, idx_hbm, out_hbm, idx_v, sem):
        wid = lax.axis_index("s") * NC + lax.axis_index("c")
        base = wid * b_per_w
        pltpu.sync_copy(idx_hbm.at[pl.ds(base, b_per_w)], idx_v)

        def chunk_body(ch, _):
            off = ch * K
            descs = []
            for q in range(K // 16):
                v = idx_v[pl.ds(off + q * 16, 16)]
                for j in range(16):
                    descs.append(
                        pltpu.async_copy(
                            table_hbm.at[v[j]],
                            out_hbm.at[base + off + q * 16 + j],
                            sem,
                        )
                    )
            for d in descs:
                d.wait()
            return ()

        lax.fori_loop(0, b_per_w // K, chunk_body, ())

    return k


def kernel(target, table):
    B = target.shape[0]
    V, D = table.shape
    k = _make_gather(V, D, B)
    return k(table, target.astype(jnp.int32))


# speedup vs baseline: 1.6909x; 1.6514x over previous
import functools

import jax
import jax.numpy as jnp
from jax import lax
from jax.experimental import pallas as pl
from jax.experimental.pallas import tpu as pltpu
from jax.experimental.pallas import tpu_sc as plsc


def _make_gather(V, D, B):
    info = plsc.get_sparse_core_info()
    NC, NS = info.num_cores, info.num_subcores
    NW = NC * NS
    assert B % (8 * NW) == 0
    b_per_w = B // NW
    K = 64
    mesh = plsc.VectorSubcoreMesh(core_axis_name="c", subcore_axis_name="s")

    @functools.partial(
        pl.kernel,
        mesh=mesh,
        out_type=jax.ShapeDtypeStruct((B, D), jnp.float32),
        scratch_types=[
            pltpu.VMEM((b_per_w,), jnp.int32),
            pltpu.VMEM((b_per_w, D), jnp.float32),
            pltpu.SemaphoreType.DMA,
            pltpu.SemaphoreType.DMA,
        ],
    )
    def k(table_hbm, idx_hbm, out_hbm, idx_v, rows_v, sem, out_sem):
        wid = lax.axis_index("s") * NC + lax.axis_index("c")
        base = wid * b_per_w
        pltpu.sync_copy(idx_hbm.at[pl.ds(base, b_per_w)], idx_v)

        def chunk_body(ch, _):
            off = ch * K
            descs = []
            for q in range(K // 16):
                v = idx_v[pl.ds(off + q * 16, 16)]
                for j in range(16):
                    r = q * 16 + j
                    descs.append(
                        pltpu.async_copy(
                            table_hbm.at[v[j]],
                            rows_v.at[off + r],
                            sem,
                        )
                    )
            for d in descs:
                d.wait()
            pltpu.async_copy(
                rows_v.at[pl.ds(off, K)],
                out_hbm.at[pl.ds(base + off, K)],
                out_sem,
            )
            return ()

        n_chunks = b_per_w // K
        lax.fori_loop(0, n_chunks, chunk_body, ())
        for _ in range(n_chunks):
            pltpu.make_async_copy(
                rows_v.at[pl.ds(0, K)],
                out_hbm.at[pl.ds(base, K)],
                out_sem,
            ).wait()

    return k


def kernel(target, table):
    B = target.shape[0]
    V, D = table.shape
    k = _make_gather(V, D, B)
    return k(table, target.astype(jnp.int32))
